# Initial kernel scaffold; baseline (speedup 1.0000x reference)
import jax
import jax.numpy as jnp
from jax.experimental import pallas as pl


def kernel(X, embed_weight, W, b):
    raise NotImplementedError("write your pallas kernel here")


# speedup vs baseline: 12.6735x; 12.6735x over previous
import functools

import jax
import jax.numpy as jnp
from jax import lax
from jax.experimental import pallas as pl
from jax.experimental.pallas import tpu as pltpu
from jax.experimental.pallas import tpu_sc as plsc

BATCH = 4096
SEQ = 200
VOCAB = 100000
EMBED = 128
CLS = 20
CP = 32
NC, NS = 2, 16
NW = NC * NS
ROWS_PER_W = BATCH // NW
CHUNK_ROWS = 16
CHUNK_TOK = CHUNK_ROWS * SEQ
NCHUNKS = ROWS_PER_W // CHUNK_ROWS
FOLD_R = 2000


def _fold_body(e_ref, w_ref, o_ref):
    o_ref[...] = lax.dot_general(
        e_ref[...], w_ref[...], (((1,), (1,)), ((), ())),
        preferred_element_type=jnp.float32)


def _fold(embed, wp):
    return pl.pallas_call(
        _fold_body,
        grid=(VOCAB // FOLD_R,),
        in_specs=[
            pl.BlockSpec((FOLD_R, EMBED), lambda i: (i, 0)),
            pl.BlockSpec((CP, EMBED), lambda i: (0, 0)),
        ],
        out_specs=pl.BlockSpec((FOLD_R, CP), lambda i: (i, 0)),
        out_shape=jax.ShapeDtypeStruct((VOCAB, CP), jnp.float32),
    )(embed, wp)


def _sc_pool(x_flat, folded, bp):
    mesh = plsc.VectorSubcoreMesh(
        core_axis_name="c", subcore_axis_name="s",
        num_cores=NC, num_subcores=NS)

    @functools.partial(
        pl.kernel,
        out_type=jax.ShapeDtypeStruct((BATCH, CP), jnp.float32),
        mesh=mesh,
        compiler_params=pltpu.CompilerParams(
            needs_layout_passes=False, use_tc_tiling_on_sc=False),
        scratch_types=[
            pltpu.VMEM((CHUNK_TOK,), jnp.int32),
            pltpu.VMEM((CHUNK_TOK, CP), jnp.float32),
            pltpu.VMEM((CHUNK_ROWS, CP), jnp.float32),
            pltpu.VMEM((CP,), jnp.float32),
            pltpu.VMEM((16,), jnp.float32),
            pltpu.SemaphoreType.DMA,
        ],
    )
    def body(x_hbm, folded_hbm, b_hbm, out_hbm, idx_v, rows_v, out_v, b_v,
             recip_v, sem):
        wid = lax.axis_index("s") * NC + lax.axis_index("c")
        pltpu.sync_copy(b_hbm, b_v)
        b0 = b_v[pl.ds(0, 16)]
        b1 = b_v[pl.ds(16, 16)]
        lanes = lax.iota(jnp.int32, 16)
        lane_base = lanes * SEQ

        def chunk_body(c, carry):
            tok_base = wid * (ROWS_PER_W * SEQ) + c * CHUNK_TOK
            pltpu.sync_copy(x_hbm.at[pl.ds(tok_base, CHUNK_TOK)], idx_v)
            pltpu.async_copy(folded_hbm.at[idx_v], rows_v, sem).wait()

            def cnt_body(t, cnt):
                v = plsc.load_gather(idx_v, [lane_base + t])
                return cnt + jnp.where(v != 0, 1, 0)

            cnt = lax.fori_loop(0, SEQ, cnt_body,
                                jnp.zeros((16,), jnp.int32))
            recip_v[...] = 1.0 / cnt.astype(jnp.float32)

            for r in range(CHUNK_ROWS):
                rb = r * SEQ

                def tok_body(t, acc):
                    a0, a1 = acc
                    a0 = a0 + rows_v[rb + t, 0:16]
                    a1 = a1 + rows_v[rb + t, 16:32]
                    return (a0, a1)

                a0, a1 = lax.fori_loop(
                    0, SEQ, tok_body,
                    (jnp.zeros((16,), jnp.float32),
                     jnp.zeros((16,), jnp.float32)))
                recip = plsc.load_gather(
                    recip_v, [jnp.full((16,), r, jnp.int32)])
                out_v[r, 0:16] = a0 * recip + b0
                out_v[r, 16:32] = a1 * recip + b1
            row_base = wid * ROWS_PER_W + c * CHUNK_ROWS
            pltpu.sync_copy(out_v, out_hbm.at[pl.ds(row_base, CHUNK_ROWS)])
            return carry

        lax.fori_loop(0, NCHUNKS, chunk_body, 0)

    return body(x_flat, folded, bp)


def kernel(X, embed_weight, W, b):
    x_flat = X.reshape(-1).astype(jnp.int32)
    wp = jnp.zeros((CP, EMBED), jnp.float32).at[:CLS].set(W)
    bp = jnp.zeros((CP,), jnp.float32).at[:CLS].set(b)
    folded = _fold(embed_weight, wp)
    out = _sc_pool(x_flat, folded, bp)
    return out[:, :CLS]


# speedup vs baseline: 19.4147x; 1.5319x over previous
import functools

import jax
import jax.numpy as jnp
from jax import lax
from jax.experimental import pallas as pl
from jax.experimental.pallas import tpu as pltpu
from jax.experimental.pallas import tpu_sc as plsc

BATCH = 4096
SEQ = 200
VOCAB = 100000
EMBED = 128
CLS = 20
CP = 32
NC, NS = 2, 16
NW = NC * NS
ROWS_PER_W = BATCH // NW
CHUNK_ROWS = 4
CHUNK_TOK = CHUNK_ROWS * SEQ
NCHUNKS = ROWS_PER_W // CHUNK_ROWS
W_TOK = ROWS_PER_W * SEQ
FOLD_R = 2000


def _fold_body(e_ref, w_ref, o_ref):
    o_ref[...] = lax.dot_general(
        e_ref[...], w_ref[...], (((1,), (1,)), ((), ())),
        preferred_element_type=jnp.float32)


def _fold(embed, wp):
    return pl.pallas_call(
        _fold_body,
        grid=(VOCAB // FOLD_R,),
        in_specs=[
            pl.BlockSpec((FOLD_R, EMBED), lambda i: (i, 0)),
            pl.BlockSpec((CP, EMBED), lambda i: (0, 0)),
        ],
        out_specs=pl.BlockSpec((FOLD_R, CP), lambda i: (i, 0)),
        out_shape=jax.ShapeDtypeStruct((VOCAB, CP), jnp.float32),
    )(embed, wp)


def _sc_pool(x_flat, folded, bp):
    mesh = plsc.VectorSubcoreMesh(
        core_axis_name="c", subcore_axis_name="s",
        num_cores=NC, num_subcores=NS)

    @functools.partial(
        pl.kernel,
        out_type=jax.ShapeDtypeStruct((BATCH, CP), jnp.float32),
        mesh=mesh,
        compiler_params=pltpu.CompilerParams(
            needs_layout_passes=False, use_tc_tiling_on_sc=False),
        scratch_types=[
            pltpu.VMEM((W_TOK,), jnp.int32),
            pltpu.VMEM((CHUNK_TOK, CP), jnp.float32),
            pltpu.VMEM((CHUNK_TOK, CP), jnp.float32),
            pltpu.VMEM((ROWS_PER_W, CP), jnp.float32),
            pltpu.VMEM((CP,), jnp.float32),
            pltpu.VMEM((16,), jnp.float32),
            pltpu.SemaphoreType.DMA,
            pltpu.SemaphoreType.DMA,
        ],
    )
    def body(x_hbm, folded_hbm, b_hbm, out_hbm, idx_v, rows_v0, rows_v1,
             out_v, b_v, recip_v, sem0, sem1):
        wid = lax.axis_index("s") * NC + lax.axis_index("c")
        pltpu.sync_copy(b_hbm, b_v)
        pltpu.sync_copy(x_hbm.at[pl.ds(wid * W_TOK, W_TOK)], idx_v)
        b0 = b_v[pl.ds(0, 16)]
        b1 = b_v[pl.ds(16, 16)]
        lanes = lax.iota(jnp.int32, 16)
        lane_base = (lanes % CHUNK_ROWS) * SEQ

        def start_gather(c, rows_v, sem):
            pltpu.make_async_copy(
                folded_hbm.at[idx_v.at[pl.ds(c * CHUNK_TOK, CHUNK_TOK)]],
                rows_v, sem).start()

        def process(c, rows_v, sem):
            pltpu.make_async_copy(
                folded_hbm.at[idx_v.at[pl.ds(c * CHUNK_TOK, CHUNK_TOK)]],
                rows_v, sem).wait()
            cbase = c * CHUNK_TOK

            def tok_body(t, carry):
                cnt = carry[0]
                v = plsc.load_gather(idx_v, [lane_base + (cbase + t)])
                cnt = cnt + jnp.where(v != 0, 1, 0)
                accs = [cnt]
                for r in range(CHUNK_ROWS):
                    accs.append(carry[1 + 2 * r] + rows_v[r * SEQ + t, 0:16])
                    accs.append(carry[2 + 2 * r] + rows_v[r * SEQ + t, 16:32])
                return tuple(accs)

            init = (jnp.zeros((16,), jnp.int32),) + tuple(
                jnp.zeros((16,), jnp.float32) for _ in range(2 * CHUNK_ROWS))
            res = lax.fori_loop(0, SEQ, tok_body, init)
            recip_v[...] = 1.0 / res[0].astype(jnp.float32)
            for r in range(CHUNK_ROWS):
                recip = plsc.load_gather(
                    recip_v, [jnp.full((16,), r, jnp.int32)])
                row = c * CHUNK_ROWS + r
                out_v[row, 0:16] = res[1 + 2 * r] * recip + b0
                out_v[row, 16:32] = res[2 + 2 * r] * recip + b1

        start_gather(0, rows_v0, sem0)

        def super_body(g, carry):
            start_gather(2 * g + 1, rows_v1, sem1)
            process(2 * g, rows_v0, sem0)

            @pl.when(g < NCHUNKS // 2 - 1)
            def _():
                start_gather(2 * g + 2, rows_v0, sem0)

            process(2 * g + 1, rows_v1, sem1)
            return carry

        lax.fori_loop(0, NCHUNKS // 2, super_body, 0)
        pltpu.sync_copy(out_v, out_hbm.at[pl.ds(wid * ROWS_PER_W,
                                                ROWS_PER_W)])

    return body(x_flat, folded, bp)


def kernel(X, embed_weight, W, b):
    x_flat = X.reshape(-1).astype(jnp.int32)
    wp = jnp.zeros((CP, EMBED), jnp.float32).at[:CLS].set(W)
    bp = jnp.zeros((CP,), jnp.float32).at[:CLS].set(b)
    folded = _fold(embed_weight, wp)
    out = _sc_pool(x_flat, folded, bp)
    return out[:, :CLS]


# speedup vs baseline: 25.1929x; 1.2976x over previous
import functools

import jax
import jax.numpy as jnp
from jax import lax
from jax.experimental import pallas as pl
from jax.experimental.pallas import tpu as pltpu
from jax.experimental.pallas import tpu_sc as plsc

BATCH = 4096
SEQ = 200
VOCAB = 100000
EMBED = 128
CLS = 20
CP = 32
NC, NS = 2, 16
NW = NC * NS
ROWS_PER_W = BATCH // NW
CHUNK_ROWS = 4
CHUNK_TOK = CHUNK_ROWS * SEQ
NCHUNKS = ROWS_PER_W // CHUNK_ROWS
W_TOK = ROWS_PER_W * SEQ
FOLD_R = 1000
CNT_R = 512


def _fold_body(e0, e1, e2, e3, w_ref, o_ref):
    def mm(e_ref):
        return lax.dot_general(
            e_ref[...], w_ref[...], (((1,), (1,)), ((), ())),
            preferred_element_type=jnp.float32)

    o_ref[...] = jnp.concatenate([mm(e0), mm(e1), mm(e2), mm(e3)], axis=1)


def _fold(embed, wp):
    q = VOCAB // 4 // FOLD_R
    return pl.pallas_call(
        _fold_body,
        grid=(q,),
        in_specs=[
            pl.BlockSpec((FOLD_R, EMBED), lambda i, j=j: (j * q + i, 0))
            for j in range(4)
        ] + [pl.BlockSpec((CP, EMBED), lambda i: (0, 0))],
        out_specs=pl.BlockSpec((FOLD_R, 4 * CP), lambda i: (i, 0)),
        out_shape=jax.ShapeDtypeStruct((VOCAB // 4, 4 * CP), jnp.float32),
    )(embed, embed, embed, embed, wp)


def _perm_body(x_ref, o_ref):
    v = x_ref[...]
    o_ref[...] = (v % (VOCAB // 4)) * 4 + v // (VOCAB // 4)


def _perm_idx(x_flat2d):
    n = x_flat2d.shape[0]
    return pl.pallas_call(
        _perm_body,
        grid=(8,),
        in_specs=[pl.BlockSpec((n // 8, 128), lambda i: (i, 0))],
        out_specs=pl.BlockSpec((n // 8, 128), lambda i: (i, 0)),
        out_shape=jax.ShapeDtypeStruct((n, 128), jnp.int32),
    )(x_flat2d)


def _cnt_body(x_ref, o_ref):
    nz = (x_ref[...] != 0).astype(jnp.float32)
    o_ref[...] = 1.0 / jnp.sum(nz, axis=1, keepdims=True)


def _recip_counts(x2d):
    return pl.pallas_call(
        _cnt_body,
        grid=(BATCH // CNT_R,),
        in_specs=[pl.BlockSpec((CNT_R, SEQ), lambda i: (i, 0))],
        out_specs=pl.BlockSpec((CNT_R, 1), lambda i: (i, 0)),
        out_shape=jax.ShapeDtypeStruct((BATCH, 1), jnp.float32),
    )(x2d)


def _sc_pool(x_flat, folded, recip, bp):
    mesh = plsc.VectorSubcoreMesh(
        core_axis_name="c", subcore_axis_name="s",
        num_cores=NC, num_subcores=NS)

    @functools.partial(
        pl.kernel,
        out_type=jax.ShapeDtypeStruct((BATCH, CP), jnp.float32),
        mesh=mesh,
        compiler_params=pltpu.CompilerParams(
            needs_layout_passes=False, use_tc_tiling_on_sc=False),
        scratch_types=[
            pltpu.VMEM((W_TOK,), jnp.int32),
            pltpu.VMEM((CHUNK_TOK, CP), jnp.float32),
            pltpu.VMEM((CHUNK_TOK, CP), jnp.float32),
            pltpu.VMEM((ROWS_PER_W, CP), jnp.float32),
            pltpu.VMEM((CP,), jnp.float32),
            pltpu.VMEM((ROWS_PER_W,), jnp.float32),
            pltpu.SemaphoreType.DMA,
            pltpu.SemaphoreType.DMA,
        ],
    )
    def body(x_hbm, folded_hbm, recip_hbm, b_hbm, out_hbm, idx_v, rows_v0,
             rows_v1, out_v, b_v, recip_v, sem0, sem1):
        wid = lax.axis_index("s") * NC + lax.axis_index("c")
        pltpu.sync_copy(b_hbm, b_v)
        pltpu.sync_copy(recip_hbm.at[pl.ds(wid * ROWS_PER_W, ROWS_PER_W)],
                        recip_v)
        pltpu.sync_copy(x_hbm.at[pl.ds(wid * W_TOK, W_TOK)], idx_v)
        b0 = b_v[pl.ds(0, 16)]
        b1 = b_v[pl.ds(16, 16)]
        zeros16i = jnp.zeros((16,), jnp.int32)

        def start_gather(c, rows_v, sem):
            pltpu.make_async_copy(
                folded_hbm.at[idx_v.at[pl.ds(c * CHUNK_TOK, CHUNK_TOK)]],
                rows_v, sem).start()

        def process(c, rows_v, sem):
            pltpu.make_async_copy(
                folded_hbm.at[idx_v.at[pl.ds(c * CHUNK_TOK, CHUNK_TOK)]],
                rows_v, sem).wait()

            def tok_body(t, carry):
                accs = []
                for r in range(CHUNK_ROWS):
                    accs.append(carry[2 * r] + rows_v[r * SEQ + t, 0:16])
                    accs.append(carry[2 * r + 1] + rows_v[r * SEQ + t, 16:32])
                return tuple(accs)

            init = tuple(
                jnp.zeros((16,), jnp.float32) for _ in range(2 * CHUNK_ROWS))
            res = lax.fori_loop(0, SEQ, tok_body, init)
            for r in range(CHUNK_ROWS):
                row = c * CHUNK_ROWS + r
                rsp = plsc.load_gather(recip_v, [zeros16i + row])
                out_v[row, 0:16] = res[2 * r] * rsp + b0
                out_v[row, 16:32] = res[2 * r + 1] * rsp + b1

        start_gather(0, rows_v0, sem0)

        def super_body(g, carry):
            start_gather(2 * g + 1, rows_v1, sem1)
            process(2 * g, rows_v0, sem0)

            @pl.when(g < NCHUNKS // 2 - 1)
            def _():
                start_gather(2 * g + 2, rows_v0, sem0)

            process(2 * g + 1, rows_v1, sem1)
            return carry

        lax.fori_loop(0, NCHUNKS // 2, super_body, 0)
        pltpu.sync_copy(out_v, out_hbm.at[pl.ds(wid * ROWS_PER_W,
                                                ROWS_PER_W)])

    return body(x_flat, folded, recip, bp)


def kernel(X, embed_weight, W, b):
    x2d = X.astype(jnp.int32)
    x_flat2d = x2d.reshape(BATCH * SEQ // 128, 128)
    wp = jnp.zeros((CP, EMBED), jnp.float32).at[:CLS].set(W)
    bp = jnp.zeros((CP,), jnp.float32).at[:CLS].set(b)
    folded = _fold(embed_weight, wp).reshape(VOCAB, CP)
    x_perm = _perm_idx(x_flat2d).reshape(BATCH * SEQ)
    recip = _recip_counts(x2d).reshape(BATCH)
    out = _sc_pool(x_perm, folded, recip, bp)
    return out[:, :CLS]
